# Initial kernel scaffold; baseline (speedup 1.0000x reference)
import jax
import jax.numpy as jnp
from jax.experimental import pallas as pl


def kernel(top_vecs, tok_struct_vec, sent_struct_vec, pos_table):
    raise NotImplementedError("write your pallas kernel here")


# speedup vs baseline: 22.3406x; 22.3406x over previous
import jax
import jax.numpy as jnp
from jax.experimental import pallas as pl


def _bcast_kernel(tbl_ref, out_ref):
    n_sents = out_ref.shape[1]
    out_ref[...] = jnp.broadcast_to(tbl_ref[:n_sents][None, :, :], out_ref.shape)


def kernel(top_vecs, tok_struct_vec, sent_struct_vec, pos_table):
    batch, n_sents = top_vecs.shape[0], top_vecs.shape[1]
    emb = pos_table.shape[1]
    b_blk = 32
    out = pl.pallas_call(
        _bcast_kernel,
        grid=(batch // b_blk,),
        in_specs=[pl.BlockSpec(pos_table.shape, lambda i: (0, 0))],
        out_specs=pl.BlockSpec((b_blk, n_sents, emb), lambda i: (i, 0, 0)),
        out_shape=jax.ShapeDtypeStruct((batch, n_sents, emb), pos_table.dtype),
    )(pos_table)
    return out
